# Initial kernel scaffold; baseline (speedup 1.0000x reference)
import jax
import jax.numpy as jnp
from jax.experimental import pallas as pl


def kernel(x, edge_index, edge_attr, batch, W1a, b1a, W1b, b1b, root1, bias1, pw1, W2a, b2a, W2b, b2b, root2, bias2, pw2, fc1W, fc1b, fc2W, fc2b, fc3W, fc3b):
    raise NotImplementedError("write your pallas kernel here")


# speedup vs baseline: 2.0759x; 2.0759x over previous
import functools
import jax
import jax.numpy as jnp
from jax import lax
from jax.experimental import pallas as pl
from jax.experimental.pallas import tpu as pltpu, tpu_sc as plsc

N = 10000
E = 20000
DIM = 128
NP = 10240
EC = 64
NCHUNK = 20
EP = 16 * NCHUNK * EC
NEG = -1.0e30


def _sc_body(pr, srcp, dstp, eap, wab, maskf, xc_out,
             acc, src_v, dst_v, ea_v, prow, msg, mb_s, mb_d,
             wab_v, accv, prv, outv, sem):
    core = lax.axis_index("c")
    wid = lax.axis_index("s")
    def splat(ref, i):
        return jnp.full((16,), ref[pl.ds(i, 16)][0])

    def splat2(ref, r, c):
        return jnp.full((16,), ref[r, pl.ds(c, 16)][0])

    def vload(ref, r, c0):
        return ref[r, pl.ds(c0, 16)]

    def vstore(ref, r, c0, val):
        ref[r, pl.ds(c0, 16)] = val

    @pl.when(core == 0)
    def _():
        def zrow(i, _):
            for g in range(5):
                vstore(accv, i, g * 16, jnp.zeros((16,), jnp.float32))
            return 0
        lax.fori_loop(0, 32, zrow, 0)

        def zcp(cc, _):
            pltpu.sync_copy(accv, acc.at[pl.ds(wid * 640 + cc * 32, 32)])
            return 0
        lax.fori_loop(0, 20, zcp, 0)

    plsc.subcore_barrier()

    @pl.when(core == 0)
    def _():
        pltpu.sync_copy(wab, wab_v)

        def chunk(ci, _):
            off = (wid * NCHUNK + ci) * EC
            pltpu.sync_copy(srcp.at[pl.ds(off, EC)], src_v)
            pltpu.sync_copy(dstp.at[pl.ds(off, EC)], dst_v)
            pltpu.sync_copy(eap.at[pl.ds(off, EC)], ea_v.at[pl.ds(0, EC)])
            pltpu.async_copy(pr.at[src_v], prow, sem).wait()
            pltpu.async_copy(maskf.at[src_v], mb_s, sem).wait()
            pltpu.async_copy(maskf.at[dst_v], mb_d, sem).wait()

            def edge(j, _):
                a = splat(ea_v, j)
                vf = splat2(mb_s, j, 0) * splat2(mb_d, j, 0)
                gc = [jnp.maximum(a * splat(wab_v, c) + splat(wab_v, 4 + c),
                                  0.0) * vf
                      for c in range(4)]
                for g in range(4):
                    accg = vf * vload(prow, j, 4 * 64 + g * 16)
                    for c in range(4):
                        accg = accg + gc[c] * vload(prow, j, c * 64 + g * 16)
                    vstore(msg, j, g * 16, accg)
                onehot0 = wab_v[pl.ds(16, 16)]
                vstore(msg, j, 64, vf * onehot0)
                return 0
            lax.fori_loop(0, EC, edge, 0)
            pltpu.sync_copy(msg, acc.at[dst_v], add=True)
            return 0
        lax.fori_loop(0, NCHUNK, chunk, 0)

    plsc.subcore_barrier()

    @pl.when(core == 0)
    def _():
        def rchunk(cc, _):
            r0 = wid * 640 + cc * 32
            pltpu.sync_copy(acc.at[pl.ds(r0, 32)], accv)
            pltpu.sync_copy(pr.at[pl.ds(r0, 32)], prv)

            def row(i, _):
                cnt = splat2(accv, i, 64)
                denom = jnp.maximum(cnt, 1.0)
                for g in range(4):
                    sg = vload(accv, i, g * 16)
                    rg = vload(prv, i, 320 + g * 16)
                    xcg = jnp.maximum(sg / denom + rg, 0.0)
                    vstore(outv, i, g * 16, xcg)
                return 0
            lax.fori_loop(0, 32, row, 0)
            pltpu.sync_copy(outv, xc_out.at[pl.ds(r0, 32)])
            return 0
        lax.fori_loop(0, 20, rchunk, 0)


def _sc_call(pr, srcp, dstp, eap, wab, maskf):
    f32 = jnp.float32
    mesh = plsc.VectorSubcoreMesh(core_axis_name="c", subcore_axis_name="s")
    k = pl.kernel(
        _sc_body,
        out_type=jax.ShapeDtypeStruct((NP, 64), f32),
        mesh=mesh,
        scratch_types=[
            pltpu.VMEM_SHARED((NP, 80), f32),
            pltpu.VMEM((EC,), jnp.int32),
            pltpu.VMEM((EC,), jnp.int32),
            pltpu.VMEM((EC + 16,), f32),
            pltpu.VMEM((EC, 384), f32),
            pltpu.VMEM((EC, 80), f32),
            pltpu.VMEM((EC, 128), f32),
            pltpu.VMEM((EC, 128), f32),
            pltpu.VMEM((32,), f32),
            pltpu.VMEM((32, 80), f32),
            pltpu.VMEM((32, 384), f32),
            pltpu.VMEM((32, 64), f32),
            pltpu.SemaphoreType.DMA,
        ],
    )
    return k(pr, srcp, dstp, eap, wab, maskf)


def _mm_body(xb, wb, ob):
    ob[...] = jnp.dot(xb[...], wb[...], preferred_element_type=jnp.float32)


def _mm_call(xaug, waug):
    K = xaug.shape[1]
    return pl.pallas_call(
        _mm_body,
        grid=(16,),
        in_specs=[pl.BlockSpec((640, K), lambda i: (i, 0)),
                  pl.BlockSpec((K, 384), lambda i: (0, 0))],
        out_specs=pl.BlockSpec((640, 384), lambda i: (i, 0)),
        out_shape=jax.ShapeDtypeStruct((NP, 384), jnp.float32),
    )(xaug, waug)


def _key(score):
    b = lax.bitcast_convert_type(score, jnp.int32)
    return b ^ (lax.shift_right_arithmetic(b, 31) & jnp.int32(0x7FFFFFFF))


def _bisect_body(k64, xcref, mfref, pwref, oref):
    s = jnp.dot(xcref[...], pwref[...], preferred_element_type=jnp.float32)
    m = mfref[...][:, 0:1]
    key = _key(s * m - (1.0 - m) * 1.0e30)

    def it(_, carry):
        lo, hi = carry
        mid = (lo >> 1) + (hi >> 1) + (lo & hi & 1)
        cnt = jnp.sum((key > mid).astype(jnp.int32))
        big = cnt >= k64
        return (jnp.where(big, mid, lo), jnp.where(big, hi, mid))

    lo0 = jnp.int32(-2147483647 - 1)
    hi0 = jnp.int32(2147483647)
    _, hi = lax.fori_loop(0, 32, it, (lo0, hi0))
    oref[...] = jnp.full((8, 128), hi, jnp.int32)


def _bisect_call(xc, mf, pwp, k):
    return pl.pallas_call(
        functools.partial(_bisect_body, int(k) * 128),
        out_shape=jax.ShapeDtypeStruct((8, 128), jnp.int32),
    )(xc, mf, pwp)


def _mask_body(xcb, mpb, pwb, vkb, xpb, mfb, mxb, smb):
    vk = vkb[0, 0]
    s = jnp.dot(xcb[...], pwb[...], preferred_element_type=jnp.float32)
    mprev = mpb[...][:, 0:1]
    sm = s * mprev - (1.0 - mprev) * 1.0e30
    maskfull = _key(sm) >= vk
    maskb = maskfull[:, 0:64]
    mf = maskb.astype(jnp.float32)
    xp = xcb[...] * jnp.tanh(sm[:, 0:64])
    xpm = xp * mf
    xpb[...] = xpm
    mfb[...] = maskfull.astype(jnp.float32)
    mxb[0, 0, :] = jnp.max(jnp.where(maskb, xp, NEG), axis=0)
    smb[0, 0, :] = jnp.sum(xpm, axis=0)


def _mask_call(xc, mfprev, pwp, vk):
    f32 = jnp.float32
    return pl.pallas_call(
        _mask_body,
        grid=(16,),
        in_specs=[pl.BlockSpec((640, 64), lambda i: (i, 0)),
                  pl.BlockSpec((640, 128), lambda i: (i, 0)),
                  pl.BlockSpec((64, 128), lambda i: (0, 0)),
                  pl.BlockSpec((8, 128), lambda i: (0, 0))],
        out_specs=[pl.BlockSpec((640, 64), lambda i: (i, 0)),
                   pl.BlockSpec((640, 128), lambda i: (i, 0)),
                   pl.BlockSpec((1, 1, 64), lambda i: (i, 0, 0)),
                   pl.BlockSpec((1, 1, 64), lambda i: (i, 0, 0))],
        out_shape=[jax.ShapeDtypeStruct((NP, 64), f32),
                   jax.ShapeDtypeStruct((NP, 128), f32),
                   jax.ShapeDtypeStruct((16, 1, 64), f32),
                   jax.ShapeDtypeStruct((16, 1, 64), f32)],
    )(xc, mfprev, pwp, vk)


def _head_body(k1, k2, mp1, sp1, mp2, sp2, w1, b1, w2, b2, w3, b3, oref):
    m1 = jnp.max(mp1[...], axis=(0, 1))[None, :]
    s1 = (jnp.sum(sp1[...], axis=(0, 1)) / k1)[None, :]
    m2 = jnp.max(mp2[...], axis=(0, 1))[None, :]
    s2 = (jnp.sum(sp2[...], axis=(0, 1)) / k2)[None, :]
    w = w1[...]
    h = m1 @ w[0:64] + s1 @ w[64:128] + m2 @ w[128:192] + s2 @ w[192:256]
    h = jnp.maximum(h + b1[0:1, :], 0.0)
    h2 = jnp.maximum(h @ w2[...] + b2[0:1, :], 0.0)
    lg = h2 @ w3[...] + b3[0:1, :]
    col = lax.broadcasted_iota(jnp.int32, (1, 128), 1)
    lgm = jnp.where(col < 2, lg, NEG)
    mx = jnp.max(lgm, axis=-1, keepdims=True)
    lse = mx + jnp.log(jnp.sum(jnp.exp(lgm - mx), axis=-1, keepdims=True))
    oref[...] = jnp.zeros((8, 128), jnp.float32)
    oref[0:1, :] = lgm - lse


def _head_call(k1, k2, mp1, sp1, mp2, sp2, w1, b1, w2, b2, w3, b3):
    return pl.pallas_call(
        functools.partial(_head_body, float(k1), float(k2)),
        out_shape=jax.ShapeDtypeStruct((8, 128), jnp.float32),
    )(mp1, sp1, mp2, sp2, w1, b1, w2, b2, w3, b3)


def kernel(x, edge_index, edge_attr, batch, W1a, b1a, W1b, b1b, root1, bias1,
           pw1, W2a, b2a, W2b, b2b, root2, bias2, pw2,
           fc1W, fc1b, fc2W, fc2b, fc3W, fc3b):
    f32 = jnp.float32
    k1 = (N + 1) // 2
    k2 = (k1 + 1) // 2
    src = edge_index[0]
    dst = edge_index[1]

    pad_e = EP - E
    srcp = jnp.concatenate([src, jnp.zeros((pad_e,), jnp.int32)])
    dstp = jnp.concatenate([dst, jnp.full((pad_e,), NP - 1, jnp.int32)])
    eap = jnp.concatenate([edge_attr[:, 0], jnp.zeros((pad_e,), f32)])

    xpad = jnp.concatenate([x, jnp.zeros((NP - N, DIM), f32)], axis=0)
    ones_col = jnp.ones((NP, 1), f32)

    def build_w(Wb, bb, root, bias, din):
        mats = [Wb[c].reshape(din, 64) for c in range(4)] + \
               [bb.reshape(din, 64), root]
        wfull = jnp.concatenate(mats, axis=1)
        brow = jnp.concatenate([jnp.zeros((1, 320), f32), bias[None, :]],
                               axis=1)
        return jnp.concatenate([wfull, brow], axis=0)

    waug1 = build_w(W1b, b1b, root1, bias1, DIM)
    waug2 = build_w(W2b, b2b, root2, bias2, 64)
    onehot = (jnp.arange(16) == 0).astype(f32)
    wab1 = jnp.concatenate([W1a[0], b1a, jnp.zeros((8,), f32), onehot])
    wab2 = jnp.concatenate([W2a[0], b2a, jnp.zeros((8,), f32), onehot])
    pw1p = jnp.tile((pw1 / jnp.linalg.norm(pw1))[:, None], (1, 128))
    pw2p = jnp.tile((pw2 / jnp.linalg.norm(pw2))[:, None], (1, 128))

    mask0 = (jnp.arange(NP) < N).astype(f32)[:, None] * jnp.ones((1, 128),
                                                                  f32)

    pr1 = _mm_call(jnp.concatenate([xpad, ones_col], axis=1), waug1)
    xc1 = _sc_call(pr1, srcp, dstp, eap, wab1, mask0)
    vk1 = _bisect_call(xc1, mask0, pw1p, k1)
    xp1m, mask1, mp1, sp1 = _mask_call(xc1, mask0, pw1p, vk1)

    pr2 = _mm_call(jnp.concatenate([xp1m, ones_col], axis=1), waug2)
    xc2 = _sc_call(pr2, srcp, dstp, eap, wab2, mask1)
    vk2 = _bisect_call(xc2, mask1, pw2p, k2)
    _, _, mp2, sp2 = _mask_call(xc2, mask1, pw2p, vk2)

    fc2Wp = jnp.concatenate([fc2W, jnp.zeros((64, 120), f32)], axis=1)
    fc3Wp = jnp.concatenate(
        [jnp.concatenate([fc3W, jnp.zeros((8, 126), f32)], axis=1),
         jnp.zeros((120, 128), f32)], axis=0)
    b1t = jnp.tile(fc1b[None, :], (8, 1))
    b2t = jnp.tile(jnp.concatenate([fc2b, jnp.zeros((120,), f32)])[None, :],
                   (8, 1))
    b3t = jnp.tile(jnp.concatenate([fc3b, jnp.zeros((126,), f32)])[None, :],
                   (8, 1))
    out = _head_call(k1, k2, mp1, sp1, mp2, sp2,
                     fc1W, b1t, fc2Wp, b2t, fc3Wp, b3t)
    return out[0:1, 0:2]


# speedup vs baseline: 2.4702x; 1.1899x over previous
import functools
import jax
import jax.numpy as jnp
from jax import lax
from jax.experimental import pallas as pl
from jax.experimental.pallas import tpu as pltpu, tpu_sc as plsc

N = 10000
E = 20000
DIM = 128
NP = 10240
EC = 64
NCHUNK = 20
EP = 16 * NCHUNK * EC
NEG = -1.0e30


def _sc_body(pr, srcp, dstp, eap, wab, maskf, xc_out,
             acc, src_v, dst_v, ea_v, prow, msg, mb_s, mb_d,
             wab_v, accv, prv, outv, sem):
    core = lax.axis_index("c")
    wid = lax.axis_index("s")
    def splat(ref, i):
        return jnp.full((16,), ref[pl.ds(i, 16)][0])

    def splat2(ref, r, c):
        return jnp.full((16,), ref[r, pl.ds(c, 16)][0])

    def vload(ref, r, c0):
        return ref[r, pl.ds(c0, 16)]

    def vstore(ref, r, c0, val):
        ref[r, pl.ds(c0, 16)] = val

    @pl.when(core == 0)
    def _():
        def zrow(i, _):
            for g in range(5):
                vstore(accv, i, g * 16, jnp.zeros((16,), jnp.float32))
            return 0
        lax.fori_loop(0, 32, zrow, 0)

        def zcp(cc, _):
            pltpu.sync_copy(accv, acc.at[pl.ds(wid * 640 + cc * 32, 32)])
            return 0
        lax.fori_loop(0, 20, zcp, 0)

    plsc.subcore_barrier()

    @pl.when(core == 0)
    def _():
        pltpu.sync_copy(wab, wab_v)

        def chunk(ci, _):
            off = (wid * NCHUNK + ci) * EC
            pltpu.sync_copy(srcp.at[pl.ds(off, EC)], src_v)
            pltpu.sync_copy(dstp.at[pl.ds(off, EC)], dst_v)
            pltpu.sync_copy(eap.at[pl.ds(off, EC)], ea_v.at[pl.ds(0, EC)])
            c1 = pltpu.async_copy(pr.at[src_v], prow, sem)
            c2 = pltpu.async_copy(maskf.at[src_v], mb_s, sem)
            c3 = pltpu.async_copy(maskf.at[dst_v], mb_d, sem)
            c1.wait()
            c2.wait()
            c3.wait()

            def edge(j, _):
                a = splat(ea_v, j)
                vf = splat2(mb_s, j, 0) * splat2(mb_d, j, 0)
                gc = [jnp.maximum(a * splat(wab_v, c) + splat(wab_v, 4 + c),
                                  0.0) * vf
                      for c in range(4)]
                for g in range(4):
                    accg = vf * vload(prow, j, 4 * 64 + g * 16)
                    for c in range(4):
                        accg = accg + gc[c] * vload(prow, j, c * 64 + g * 16)
                    vstore(msg, j, g * 16, accg)
                onehot0 = wab_v[pl.ds(16, 16)]
                vstore(msg, j, 64, vf * onehot0)
                return 0
            lax.fori_loop(0, EC, edge, 0)
            pltpu.sync_copy(msg, acc.at[dst_v], add=True)
            return 0
        lax.fori_loop(0, NCHUNK, chunk, 0)

    plsc.subcore_barrier()

    @pl.when(core == 0)
    def _():
        def rchunk(cc, _):
            r0 = wid * 640 + cc * 32
            pltpu.sync_copy(acc.at[pl.ds(r0, 32)], accv)
            pltpu.sync_copy(pr.at[pl.ds(r0, 32)], prv)

            def row(i, _):
                cnt = splat2(accv, i, 64)
                denom = jnp.maximum(cnt, 1.0)
                for g in range(4):
                    sg = vload(accv, i, g * 16)
                    rg = vload(prv, i, 320 + g * 16)
                    xcg = jnp.maximum(sg / denom + rg, 0.0)
                    vstore(outv, i, g * 16, xcg)
                return 0
            lax.fori_loop(0, 32, row, 0)
            pltpu.sync_copy(outv, xc_out.at[pl.ds(r0, 32)])
            return 0
        lax.fori_loop(0, 20, rchunk, 0)


def _sc_call(pr, srcp, dstp, eap, wab, maskf):
    f32 = jnp.float32
    mesh = plsc.VectorSubcoreMesh(core_axis_name="c", subcore_axis_name="s")
    k = pl.kernel(
        _sc_body,
        out_type=jax.ShapeDtypeStruct((NP, 64), f32),
        mesh=mesh,
        scratch_types=[
            pltpu.VMEM_SHARED((NP, 80), f32),
            pltpu.VMEM((EC,), jnp.int32),
            pltpu.VMEM((EC,), jnp.int32),
            pltpu.VMEM((EC + 16,), f32),
            pltpu.VMEM((EC, 384), f32),
            pltpu.VMEM((EC, 80), f32),
            pltpu.VMEM((EC, 128), f32),
            pltpu.VMEM((EC, 128), f32),
            pltpu.VMEM((32,), f32),
            pltpu.VMEM((32, 80), f32),
            pltpu.VMEM((32, 384), f32),
            pltpu.VMEM((32, 64), f32),
            pltpu.SemaphoreType.DMA,
        ],
    )
    return k(pr, srcp, dstp, eap, wab, maskf)


def _mm_body(xb, wb, ob):
    ob[...] = jnp.dot(xb[...], wb[...], preferred_element_type=jnp.float32)


def _mm_call(xaug, waug):
    K = xaug.shape[1]
    return pl.pallas_call(
        _mm_body,
        grid=(16,),
        in_specs=[pl.BlockSpec((640, K), lambda i: (i, 0)),
                  pl.BlockSpec((K, 384), lambda i: (0, 0))],
        out_specs=pl.BlockSpec((640, 384), lambda i: (i, 0)),
        out_shape=jax.ShapeDtypeStruct((NP, 384), jnp.float32),
    )(xaug, waug)


def _key(score):
    b = lax.bitcast_convert_type(score, jnp.int32)
    return b ^ (lax.shift_right_arithmetic(b, 31) & jnp.int32(0x7FFFFFFF))


def _bisect_body(k64, xcref, mfref, pwref, oref):
    s = jnp.dot(xcref[...], pwref[...], preferred_element_type=jnp.float32)
    m = mfref[...][:, 0:1]
    key = _key(s * m - (1.0 - m) * 1.0e30)

    def it(_, carry):
        lo, hi = carry
        mid = (lo >> 1) + (hi >> 1) + (lo & hi & 1)
        cnt = jnp.sum((key > mid).astype(jnp.int32))
        big = cnt >= k64
        return (jnp.where(big, mid, lo), jnp.where(big, hi, mid))

    lo0 = jnp.int32(-2147483647 - 1)
    hi0 = jnp.int32(2147483647)
    _, hi = lax.fori_loop(0, 32, it, (lo0, hi0))
    oref[...] = jnp.full((8, 128), hi, jnp.int32)


def _bisect_call(xc, mf, pwp, k):
    return pl.pallas_call(
        functools.partial(_bisect_body, int(k) * 128),
        out_shape=jax.ShapeDtypeStruct((8, 128), jnp.int32),
    )(xc, mf, pwp)


def _mask_body(xcb, mpb, pwb, vkb, xpb, mfb, mxb, smb):
    vk = vkb[0, 0]
    s = jnp.dot(xcb[...], pwb[...], preferred_element_type=jnp.float32)
    mprev = mpb[...][:, 0:1]
    sm = s * mprev - (1.0 - mprev) * 1.0e30
    maskfull = _key(sm) >= vk
    maskb = maskfull[:, 0:64]
    mf = maskb.astype(jnp.float32)
    xp = xcb[...] * jnp.tanh(sm[:, 0:64])
    xpm = xp * mf
    xpb[...] = xpm
    mfb[...] = maskfull.astype(jnp.float32)
    mxb[0, 0, :] = jnp.max(jnp.where(maskb, xp, NEG), axis=0)
    smb[0, 0, :] = jnp.sum(xpm, axis=0)


def _mask_call(xc, mfprev, pwp, vk):
    f32 = jnp.float32
    return pl.pallas_call(
        _mask_body,
        grid=(16,),
        in_specs=[pl.BlockSpec((640, 64), lambda i: (i, 0)),
                  pl.BlockSpec((640, 128), lambda i: (i, 0)),
                  pl.BlockSpec((64, 128), lambda i: (0, 0)),
                  pl.BlockSpec((8, 128), lambda i: (0, 0))],
        out_specs=[pl.BlockSpec((640, 64), lambda i: (i, 0)),
                   pl.BlockSpec((640, 128), lambda i: (i, 0)),
                   pl.BlockSpec((1, 1, 64), lambda i: (i, 0, 0)),
                   pl.BlockSpec((1, 1, 64), lambda i: (i, 0, 0))],
        out_shape=[jax.ShapeDtypeStruct((NP, 64), f32),
                   jax.ShapeDtypeStruct((NP, 128), f32),
                   jax.ShapeDtypeStruct((16, 1, 64), f32),
                   jax.ShapeDtypeStruct((16, 1, 64), f32)],
    )(xc, mfprev, pwp, vk)


def _head_body(k1, k2, mp1, sp1, mp2, sp2, w1, b1, w2, b2, w3, b3, oref):
    m1 = jnp.max(mp1[...], axis=(0, 1))[None, :]
    s1 = (jnp.sum(sp1[...], axis=(0, 1)) / k1)[None, :]
    m2 = jnp.max(mp2[...], axis=(0, 1))[None, :]
    s2 = (jnp.sum(sp2[...], axis=(0, 1)) / k2)[None, :]
    w = w1[...]
    h = m1 @ w[0:64] + s1 @ w[64:128] + m2 @ w[128:192] + s2 @ w[192:256]
    h = jnp.maximum(h + b1[0:1, :], 0.0)
    h2 = jnp.maximum(h @ w2[...] + b2[0:1, :], 0.0)
    lg = h2 @ w3[...] + b3[0:1, :]
    col = lax.broadcasted_iota(jnp.int32, (1, 128), 1)
    lgm = jnp.where(col < 2, lg, NEG)
    mx = jnp.max(lgm, axis=-1, keepdims=True)
    lse = mx + jnp.log(jnp.sum(jnp.exp(lgm - mx), axis=-1, keepdims=True))
    oref[...] = jnp.zeros((8, 128), jnp.float32)
    oref[0:1, :] = lgm - lse


def _head_call(k1, k2, mp1, sp1, mp2, sp2, w1, b1, w2, b2, w3, b3):
    return pl.pallas_call(
        functools.partial(_head_body, float(k1), float(k2)),
        out_shape=jax.ShapeDtypeStruct((8, 128), jnp.float32),
    )(mp1, sp1, mp2, sp2, w1, b1, w2, b2, w3, b3)


def kernel(x, edge_index, edge_attr, batch, W1a, b1a, W1b, b1b, root1, bias1,
           pw1, W2a, b2a, W2b, b2b, root2, bias2, pw2,
           fc1W, fc1b, fc2W, fc2b, fc3W, fc3b):
    f32 = jnp.float32
    k1 = (N + 1) // 2
    k2 = (k1 + 1) // 2
    src = edge_index[0]
    dst = edge_index[1]

    pad_e = EP - E
    srcp = jnp.concatenate([src, jnp.zeros((pad_e,), jnp.int32)])
    dstp = jnp.concatenate([dst, jnp.full((pad_e,), NP - 1, jnp.int32)])
    eap = jnp.concatenate([edge_attr[:, 0], jnp.zeros((pad_e,), f32)])

    xpad = jnp.concatenate([x, jnp.zeros((NP - N, DIM), f32)], axis=0)
    ones_col = jnp.ones((NP, 1), f32)

    def build_w(Wb, bb, root, bias, din):
        mats = [Wb[c].reshape(din, 64) for c in range(4)] + \
               [bb.reshape(din, 64), root]
        wfull = jnp.concatenate(mats, axis=1)
        brow = jnp.concatenate([jnp.zeros((1, 320), f32), bias[None, :]],
                               axis=1)
        return jnp.concatenate([wfull, brow], axis=0)

    waug1 = build_w(W1b, b1b, root1, bias1, DIM)
    waug2 = build_w(W2b, b2b, root2, bias2, 64)
    onehot = (jnp.arange(16) == 0).astype(f32)
    wab1 = jnp.concatenate([W1a[0], b1a, jnp.zeros((8,), f32), onehot])
    wab2 = jnp.concatenate([W2a[0], b2a, jnp.zeros((8,), f32), onehot])
    pw1p = jnp.tile((pw1 / jnp.linalg.norm(pw1))[:, None], (1, 128))
    pw2p = jnp.tile((pw2 / jnp.linalg.norm(pw2))[:, None], (1, 128))

    mask0 = (jnp.arange(NP) < N).astype(f32)[:, None] * jnp.ones((1, 128),
                                                                  f32)

    pr1 = _mm_call(jnp.concatenate([xpad, ones_col], axis=1), waug1)
    xc1 = _sc_call(pr1, srcp, dstp, eap, wab1, mask0)
    vk1 = _bisect_call(xc1, mask0, pw1p, k1)
    xp1m, mask1, mp1, sp1 = _mask_call(xc1, mask0, pw1p, vk1)

    pr2 = _mm_call(jnp.concatenate([xp1m, ones_col], axis=1), waug2)
    xc2 = _sc_call(pr2, srcp, dstp, eap, wab2, mask1)
    vk2 = _bisect_call(xc2, mask1, pw2p, k2)
    _, _, mp2, sp2 = _mask_call(xc2, mask1, pw2p, vk2)

    fc2Wp = jnp.concatenate([fc2W, jnp.zeros((64, 120), f32)], axis=1)
    fc3Wp = jnp.concatenate(
        [jnp.concatenate([fc3W, jnp.zeros((8, 126), f32)], axis=1),
         jnp.zeros((120, 128), f32)], axis=0)
    b1t = jnp.tile(fc1b[None, :], (8, 1))
    b2t = jnp.tile(jnp.concatenate([fc2b, jnp.zeros((120,), f32)])[None, :],
                   (8, 1))
    b3t = jnp.tile(jnp.concatenate([fc3b, jnp.zeros((126,), f32)])[None, :],
                   (8, 1))
    out = _head_call(k1, k2, mp1, sp1, mp2, sp2,
                     fc1W, b1t, fc2Wp, b2t, fc3Wp, b3t)
    return out[0:1, 0:2]
